# baseline (device time: 20043 ns/iter reference)
import jax
import jax.numpy as jnp
from jax import lax
from jax.experimental import pallas as pl
from jax.experimental.pallas import tpu as pltpu

_MESH = pl.DeviceIdType.MESH


def kernel(partial, resid, gamma):
    m, d = resid.shape
    qm = m // 4
    hm = qm // 2
    p2 = partial.reshape(m, d)
    g2 = gamma.reshape(1, d)

    def body(p_ref, r_ref, g_ref, o_ref, comm_ref, send_sems, recv_sems):
        X = lax.axis_index("x")
        Y = lax.axis_index("y")
        Z = lax.axis_index("z")
        y_nbr = (X, 1 - Y, Z)
        x_nbr = (1 - X, Y, Z)
        z_nbr = (X, Y, 1 - Z)

        myQ = 2 * X + Z
        qx = 2 * (1 - X) + Z
        qz = 2 * X + (1 - Z)
        rows_my = pl.ds(myQ * qm, qm)

        barrier_sem = pltpu.get_barrier_semaphore()
        for nbr in (y_nbr, x_nbr, z_nbr):
            pl.semaphore_signal(
                barrier_sem, inc=1, device_id=nbr, device_id_type=_MESH
            )
        pl.semaphore_wait(barrier_sem, 3)

        p1 = pltpu.make_async_remote_copy(
            src_ref=p_ref.at[rows_my, :],
            dst_ref=comm_ref,
            send_sem=send_sems.at[0],
            recv_sem=recv_sems.at[0],
            device_id=y_nbr,
            device_id_type=_MESH,
        )
        p1.start()
        p1.wait_recv()

        y = p_ref[rows_my, :] + comm_ref[...] + r_ref[rows_my, :]
        ms = jnp.mean(y * y, axis=-1, keepdims=True)
        o_ref[rows_my, :] = y * lax.rsqrt(ms + 1e-6) * g_ref[...]

        a_x = pltpu.make_async_remote_copy(
            src_ref=o_ref.at[rows_my, :],
            dst_ref=o_ref.at[rows_my, :],
            send_sem=send_sems.at[1],
            recv_sem=recv_sems.at[1],
            device_id=x_nbr,
            device_id_type=_MESH,
        )
        a_z = pltpu.make_async_remote_copy(
            src_ref=o_ref.at[rows_my, :],
            dst_ref=o_ref.at[rows_my, :],
            send_sem=send_sems.at[2],
            recv_sem=recv_sems.at[2],
            device_id=z_nbr,
            device_id_type=_MESH,
        )
        a_x.start()
        a_z.start()

        a_z.wait_recv()
        rows_fx = pl.ds(qz * qm, hm)
        b_x = pltpu.make_async_remote_copy(
            src_ref=o_ref.at[rows_fx, :],
            dst_ref=o_ref.at[rows_fx, :],
            send_sem=send_sems.at[3],
            recv_sem=recv_sems.at[3],
            device_id=x_nbr,
            device_id_type=_MESH,
        )
        b_x.start()

        a_x.wait_recv()
        rows_fz = pl.ds(qx * qm + hm, hm)
        b_z = pltpu.make_async_remote_copy(
            src_ref=o_ref.at[rows_fz, :],
            dst_ref=o_ref.at[rows_fz, :],
            send_sem=send_sems.at[4],
            recv_sem=recv_sems.at[4],
            device_id=z_nbr,
            device_id_type=_MESH,
        )
        b_z.start()

        b_x.wait_recv()
        b_z.wait_recv()
        for r in (p1, a_x, a_z, b_x, b_z):
            r.wait_send()

    return pl.pallas_call(
        body,
        out_shape=jax.ShapeDtypeStruct((m, d), jnp.float32),
        in_specs=[
            pl.BlockSpec(memory_space=pltpu.VMEM),
            pl.BlockSpec(memory_space=pltpu.VMEM),
            pl.BlockSpec(memory_space=pltpu.VMEM),
        ],
        out_specs=pl.BlockSpec(memory_space=pltpu.VMEM),
        scratch_shapes=[
            pltpu.VMEM((qm, d), jnp.float32),
            pltpu.SemaphoreType.DMA((5,)),
            pltpu.SemaphoreType.DMA((5,)),
        ],
        compiler_params=pltpu.CompilerParams(collective_id=0),
    )(p2, resid, g2)


# device time: 18607 ns/iter; 1.0772x vs baseline; 1.0772x over previous
import jax
import jax.numpy as jnp
from jax import lax
from jax.experimental import pallas as pl
from jax.experimental.pallas import tpu as pltpu

_MESH = pl.DeviceIdType.MESH


def kernel(partial, resid, gamma):
    m, d = resid.shape
    qm = m // 4
    hm = qm // 2
    p2 = partial.reshape(m, d)
    g2 = gamma.reshape(1, d)

    def body(p_ref, r_ref, g_ref, o_ref, comm_ref, send_sems, recv_sems):
        X = lax.axis_index("x")
        Y = lax.axis_index("y")
        Z = lax.axis_index("z")
        y_nbr = (X, 1 - Y, Z)
        x_nbr = (1 - X, Y, Z)
        z_nbr = (X, Y, 1 - Z)

        myQ = 2 * X + Z
        qx = 2 * (1 - X) + Z
        qz = 2 * X + (1 - Z)

        barrier_sem = pltpu.get_barrier_semaphore()
        for nbr in (y_nbr, x_nbr, z_nbr):
            pl.semaphore_signal(
                barrier_sem, inc=1, device_id=nbr, device_id_type=_MESH
            )
        pl.semaphore_wait(barrier_sem, 3)

        p1 = []
        for c in range(2):
            r = pltpu.make_async_remote_copy(
                src_ref=p_ref.at[pl.ds(myQ * qm + c * hm, hm), :],
                dst_ref=comm_ref.at[pl.ds(c * hm, hm), :],
                send_sem=send_sems.at[c],
                recv_sem=recv_sems.at[c],
                device_id=y_nbr,
                device_id_type=_MESH,
            )
            r.start()
            p1.append(r)

        a_x, a_z = [], []
        for c in range(2):
            p1[c].wait_recv()
            rows = pl.ds(myQ * qm + c * hm, hm)
            y = p_ref[rows, :] + comm_ref[pl.ds(c * hm, hm), :] + r_ref[rows, :]
            ms = jnp.mean(y * y, axis=-1, keepdims=True)
            o_ref[rows, :] = y * lax.rsqrt(ms + 1e-6) * g_ref[...]
            for lst, nbr, base in ((a_x, x_nbr, 2), (a_z, z_nbr, 4)):
                r = pltpu.make_async_remote_copy(
                    src_ref=o_ref.at[rows, :],
                    dst_ref=o_ref.at[rows, :],
                    send_sem=send_sems.at[base + c],
                    recv_sem=recv_sems.at[base + c],
                    device_id=nbr,
                    device_id_type=_MESH,
                )
                r.start()
                lst.append(r)

        a_z[0].wait_recv()
        rows_fx = pl.ds(qz * qm, hm)
        b_x = pltpu.make_async_remote_copy(
            src_ref=o_ref.at[rows_fx, :],
            dst_ref=o_ref.at[rows_fx, :],
            send_sem=send_sems.at[6],
            recv_sem=recv_sems.at[6],
            device_id=x_nbr,
            device_id_type=_MESH,
        )
        b_x.start()

        a_x[1].wait_recv()
        rows_fz = pl.ds(qx * qm + hm, hm)
        b_z = pltpu.make_async_remote_copy(
            src_ref=o_ref.at[rows_fz, :],
            dst_ref=o_ref.at[rows_fz, :],
            send_sem=send_sems.at[7],
            recv_sem=recv_sems.at[7],
            device_id=z_nbr,
            device_id_type=_MESH,
        )
        b_z.start()

        a_x[0].wait_recv()
        a_z[1].wait_recv()
        b_x.wait_recv()
        b_z.wait_recv()
        for r in p1 + a_x + a_z + [b_x, b_z]:
            r.wait_send()

    return pl.pallas_call(
        body,
        out_shape=jax.ShapeDtypeStruct((m, d), jnp.float32),
        in_specs=[
            pl.BlockSpec(memory_space=pltpu.VMEM),
            pl.BlockSpec(memory_space=pltpu.VMEM),
            pl.BlockSpec(memory_space=pltpu.VMEM),
        ],
        out_specs=pl.BlockSpec(memory_space=pltpu.VMEM),
        scratch_shapes=[
            pltpu.VMEM((qm, d), jnp.float32),
            pltpu.SemaphoreType.DMA((8,)),
            pltpu.SemaphoreType.DMA((8,)),
        ],
        compiler_params=pltpu.CompilerParams(collective_id=0),
    )(p2, resid, g2)
